# baseline (speedup 1.0000x reference)
import functools

import jax
import jax.numpy as jnp
from jax import lax
from jax.experimental import pallas as pl
from jax.experimental.pallas import tpu as pltpu
from jax.experimental.pallas import tpu_sc as plsc

_LANES = 16


def _build(n_rows: int, n_cols: int, n_workers: int):
    rows_per = n_rows // n_workers
    per = rows_per * n_cols
    assert rows_per * n_workers == n_rows
    tail = n_cols - _LANES
    assert 0 < tail <= _LANES and per % 8 == 0
    mesh = plsc.VectorSubcoreMesh(core_axis_name="c", subcore_axis_name="s")

    @functools.partial(
        pl.kernel,
        out_type=jax.ShapeDtypeStruct((n_rows * n_cols,), jnp.float32),
        mesh=mesh,
        scratch_types=[
            pltpu.VMEM((_LANES,), jnp.float32),
            pltpu.VMEM((n_cols, rows_per), jnp.int32),
            pltpu.VMEM((per,), jnp.float32),
        ],
        compiler_params=pltpu.CompilerParams(needs_layout_passes=False),
    )
    def run(xt_hbm, w_hbm, out_hbm, tab, xts, ov):
        wid = lax.axis_index("s") * 2 + lax.axis_index("c")
        row0 = wid * rows_per
        pltpu.sync_copy(w_hbm, tab)
        t = jnp.exp(tab[...])
        pltpu.sync_copy(xt_hbm.at[:, pl.ds(row0, rows_per)], xts)
        col_a = lax.iota(jnp.int32, _LANES)
        col_b = col_a + tail

        @plsc.parallel_loop(0, rows_per, 1, unroll=8)
        def body(r):
            rv = jnp.full((_LANES,), r, jnp.int32)
            ia = plsc.load_gather(xts, [col_a, rv])
            ib = plsc.load_gather(xts, [col_b, rv])
            ov[pl.ds(r * n_cols, _LANES)] = t.at[ia].get(
                mode="promise_in_bounds")
            ov[pl.ds(r * n_cols + tail, _LANES)] = t.at[ib].get(
                mode="promise_in_bounds")

        pltpu.sync_copy(ov, out_hbm.at[pl.ds(row0 * n_cols, per)])

    return run


def kernel(x, weight):
    n_rows, n_cols = x.shape
    wpad = jnp.pad(weight.astype(jnp.float32), (0, _LANES - weight.shape[0]))
    return _build(n_rows, n_cols, 32)(x.T.astype(jnp.int32), wpad)


# speedup vs baseline: 1.0225x; 1.0225x over previous
import functools

import jax
import jax.numpy as jnp
from jax import lax
from jax.experimental import pallas as pl
from jax.experimental.pallas import tpu as pltpu
from jax.experimental.pallas import tpu_sc as plsc

_LANES = 16


def _build(n_rows: int, n_cols: int, n_workers: int):
    rows_per = n_rows // n_workers
    per = rows_per * n_cols
    assert rows_per * n_workers == n_rows
    tail = n_cols - _LANES
    assert 0 < tail <= _LANES and per % 8 == 0
    mesh = plsc.VectorSubcoreMesh(core_axis_name="c", subcore_axis_name="s")

    @functools.partial(
        pl.kernel,
        out_type=jax.ShapeDtypeStruct((n_rows * n_cols,), jnp.float32),
        mesh=mesh,
        scratch_types=[
            pltpu.VMEM((_LANES,), jnp.float32),
            pltpu.VMEM((n_cols, rows_per + 1), jnp.int32),
            pltpu.VMEM((per,), jnp.float32),
        ],
        compiler_params=pltpu.CompilerParams(needs_layout_passes=False),
    )
    def run(xt_hbm, w_hbm, out_hbm, tab, xts, ov):
        wid = lax.axis_index("s") * 2 + lax.axis_index("c")
        row0 = wid * rows_per
        pltpu.sync_copy(w_hbm, tab.at[pl.ds(0, 3)])
        t = jnp.exp(tab[...])
        pltpu.sync_copy(xt_hbm.at[:, pl.ds(row0, rows_per)],
                        xts.at[:, pl.ds(0, rows_per)])
        col_a = lax.iota(jnp.int32, _LANES)
        col_b = col_a + tail

        @plsc.parallel_loop(0, rows_per, 1, unroll=8)
        def body(r):
            rv = jnp.full((_LANES,), r, jnp.int32)
            ia = plsc.load_gather(xts, [col_a, rv])
            ib = plsc.load_gather(xts, [col_b, rv])
            ov[pl.ds(r * n_cols, _LANES)] = t.at[ia].get(
                mode="promise_in_bounds")
            ov[pl.ds(r * n_cols + tail, _LANES)] = t.at[ib].get(
                mode="promise_in_bounds")

        pltpu.sync_copy(ov, out_hbm.at[pl.ds(row0 * n_cols, per)])

    return run


def kernel(x, weight):
    n_rows, n_cols = x.shape
    return _build(n_rows, n_cols, 32)(
        x.T.astype(jnp.int32), weight.astype(jnp.float32))


# speedup vs baseline: 1.3361x; 1.3067x over previous
import functools

import jax
import jax.numpy as jnp
from jax import lax
from jax.experimental import pallas as pl
from jax.experimental.pallas import tpu as pltpu
from jax.experimental.pallas import tpu_sc as plsc

_LANES = 16


def _build(n_rows: int, n_cols: int, n_workers: int):
    rows_per = n_rows // n_workers
    per = rows_per * n_cols
    n_chunks = rows_per // _LANES
    assert rows_per * n_workers == n_rows and n_chunks * _LANES == rows_per
    mesh = plsc.VectorSubcoreMesh(core_axis_name="c", subcore_axis_name="s")

    @functools.partial(
        pl.kernel,
        out_type=jax.ShapeDtypeStruct((n_rows * n_cols,), jnp.float32),
        mesh=mesh,
        scratch_types=[
            pltpu.VMEM((_LANES,), jnp.float32),
            pltpu.VMEM((n_cols, rows_per), jnp.int32),
            pltpu.VMEM((per,), jnp.float32),
        ],
        compiler_params=pltpu.CompilerParams(needs_layout_passes=False),
    )
    def run(xt_hbm, w_hbm, out_hbm, tab, xts, ov):
        wid = lax.axis_index("s") * 2 + lax.axis_index("c")
        row0 = wid * rows_per
        pltpu.sync_copy(w_hbm, tab.at[pl.ds(0, 3)])
        t = jnp.exp(tab[...])
        pltpu.sync_copy(xt_hbm.at[:, pl.ds(row0, rows_per)], xts)
        addr_col = lax.iota(jnp.int32, _LANES) * n_cols

        @plsc.parallel_loop(0, n_chunks, 1)
        def body(k):
            base = k * (_LANES * n_cols)
            for j in range(n_cols):
                idx = xts[j, pl.ds(k * _LANES, _LANES)]
                vals = t.at[idx].get(mode="promise_in_bounds")
                plsc.store_scatter(ov, [addr_col + (base + j)], vals)

        pltpu.sync_copy(ov, out_hbm.at[pl.ds(row0 * n_cols, per)])

    return run


def kernel(x, weight):
    n_rows, n_cols = x.shape
    return _build(n_rows, n_cols, 32)(
        x.T.astype(jnp.int32), weight.astype(jnp.float32))
